# baseline (speedup 1.0000x reference)
import functools

import jax
import jax.numpy as jnp
from jax import lax
from jax.experimental import pallas as pl
from jax.experimental.pallas import tpu as pltpu
from jax.experimental.pallas import tpu_sc as plsc

D_MODEL = 768
D_FF = 3072
N_EXP = 64
N_TOK = 8192
TB = 512
N_TB = N_TOK // TB
B = 64
G = N_TOK // B + N_EXP
NW = 32
ROWS_W = N_TOK // NW
CH = 8
CHROWS = ROWS_W // CH
NBUF = 4


def _route_body(x_ref, wr_ref, br_ref, p_ref, be_ref, nu_ref,
                cnt_s, eid_s, rank_s):
    ph = pl.program_id(0)
    j = pl.program_id(1)

    @pl.when(jnp.logical_and(ph == 0, j == 0))
    def _():
        cnt_s[...] = jnp.zeros_like(cnt_s)

    @pl.when(ph == 0)
    def _():
        logits = jnp.dot(x_ref[...], wr_ref[...],
                         preferred_element_type=jnp.float32) + br_ref[...]
        m = jnp.max(logits, axis=1, keepdims=True)
        lane = lax.broadcasted_iota(jnp.int32, (TB, N_EXP), 1)
        eid = jnp.min(jnp.where(logits == m, lane, N_EXP), axis=1,
                      keepdims=True)
        onehot = (eid == lane).astype(jnp.float32)
        row = lax.broadcasted_iota(jnp.int32, (TB, TB), 0)
        col = lax.broadcasted_iota(jnp.int32, (TB, TB), 1)
        ltri = (col < row).astype(jnp.float32)
        prev = cnt_s[...]
        cum = jnp.dot(ltri, onehot, preferred_element_type=jnp.float32) + prev
        rank = jnp.sum(cum * onehot, axis=1, keepdims=True)
        cnt_s[...] = prev + jnp.sum(onehot, axis=0, keepdims=True)
        eid_s[pl.ds(j * TB, TB), :] = eid
        rank_s[pl.ds(j * TB, TB), :] = rank.astype(jnp.int32)

    @pl.when(ph == 1)
    def _():
        cnt = cnt_s[...].astype(jnp.int32)
        nblk = lax.shift_right_logical(cnt + (B - 1),
                                       B.bit_length() - 1)
        ur = lax.broadcasted_iota(jnp.int32, (N_EXP, N_EXP), 0)
        uc = lax.broadcasted_iota(jnp.int32, (N_EXP, N_EXP), 1)
        utri = (ur < uc).astype(jnp.float32)
        bstart = jnp.dot(nblk.astype(jnp.float32), utri,
                         preferred_element_type=jnp.float32)
        eid = eid_s[pl.ds(j * TB, TB), :]
        lane = lax.broadcasted_iota(jnp.int32, (TB, N_EXP), 1)
        onehot = (eid == lane).astype(jnp.float32)
        slot = jnp.sum(onehot * (bstart * B), axis=1, keepdims=True)
        p_ref[...] = slot.astype(jnp.int32) + rank_s[pl.ds(j * TB, TB), :]

        @pl.when(j == 0)
        def _():
            gcol = lax.broadcasted_iota(jnp.int32, (G, N_EXP), 0)
            nge = jnp.sum((gcol >= bstart.astype(jnp.int32)).astype(jnp.int32),
                          axis=1, keepdims=True)
            be_ref[...] = nge - 1
            nu_ref[...] = jnp.sum(nblk, axis=1, keepdims=True)


_route = pl.pallas_call(
    _route_body,
    grid=(2, N_TB),
    in_specs=[
        pl.BlockSpec((TB, D_MODEL),
                     lambda ph, j: (jnp.where(ph == 0, j, N_TB - 1), 0)),
        pl.BlockSpec((D_MODEL, N_EXP), lambda ph, j: (0, 0)),
        pl.BlockSpec((1, N_EXP), lambda ph, j: (0, 0)),
    ],
    out_specs=[
        pl.BlockSpec((TB, 1), lambda ph, j: (j, 0)),
        pl.BlockSpec((G, 1), lambda ph, j: (0, 0)),
        pl.BlockSpec((1, 1), lambda ph, j: (0, 0)),
    ],
    out_shape=[
        jax.ShapeDtypeStruct((N_TOK, 1), jnp.int32),
        jax.ShapeDtypeStruct((G, 1), jnp.int32),
        jax.ShapeDtypeStruct((1, 1), jnp.int32),
    ],
    scratch_shapes=[
        pltpu.VMEM((1, N_EXP), jnp.float32),
        pltpu.VMEM((N_TOK, 1), jnp.int32),
        pltpu.VMEM((N_TOK, 1), jnp.int32),
    ],
)


@functools.cache
def _sc_kernels():
    mesh = plsc.VectorSubcoreMesh(core_axis_name="c", subcore_axis_name="s")
    scratch = (
        [pltpu.VMEM((CH, CHROWS), jnp.int32)]
        + [pltpu.VMEM((CHROWS, D_MODEL), jnp.float32) for _ in range(NBUF)]
        + [pltpu.SemaphoreType.DMA, pltpu.SemaphoreType.DMA]
    )


    @functools.partial(
        pl.kernel,
        mesh=mesh,
        out_type=jax.ShapeDtypeStruct((G * B, D_MODEL), jnp.float32),
        scratch_types=scratch,
    )
    def dispatch(x_hbm, p_hbm, xg_hbm, idx_v, *rest):
        bufs, (sem_r, sem_w) = rest[:NBUF], rest[NBUF:]
        wid = lax.axis_index("s") * 2 + lax.axis_index("c")
        pltpu.sync_copy(p_hbm.at[wid], idx_v)
        rd = [None] * CH
        wr = [None] * CH
        for j in range(min(NBUF, CH)):
            base = wid * ROWS_W + j * CHROWS
            rd[j] = pltpu.async_copy(x_hbm.at[pl.ds(base, CHROWS)],
                                     bufs[j % NBUF], sem_r)
        for j in range(CH):
            rd[j].wait()
            wr[j] = pltpu.async_copy(bufs[j % NBUF], xg_hbm.at[idx_v.at[j]],
                                     sem_w)
            nxt = j + NBUF
            if nxt < CH:
                wr[j].wait()
                base = wid * ROWS_W + nxt * CHROWS
                rd[nxt] = pltpu.async_copy(x_hbm.at[pl.ds(base, CHROWS)],
                                           bufs[nxt % NBUF], sem_r)
        for j in range(CH - NBUF, CH):
            if wr[j] is not None:
                wr[j].wait()

    @functools.partial(
        pl.kernel,
        mesh=mesh,
        out_type=jax.ShapeDtypeStruct((N_TOK, D_MODEL), jnp.float32),
        scratch_types=scratch,
    )
    def combine(y_hbm, p_hbm, out_hbm, idx_v, *rest):
        bufs, (sem_r, sem_w) = rest[:NBUF], rest[NBUF:]
        wid = lax.axis_index("s") * 2 + lax.axis_index("c")
        pltpu.sync_copy(p_hbm.at[wid], idx_v)
        rd = [None] * CH
        wr = [None] * CH
        for j in range(min(NBUF, CH)):
            rd[j] = pltpu.async_copy(y_hbm.at[idx_v.at[j]], bufs[j % NBUF],
                                     sem_r)
        for j in range(CH):
            rd[j].wait()
            base = wid * ROWS_W + j * CHROWS
            wr[j] = pltpu.async_copy(bufs[j % NBUF],
                                     out_hbm.at[pl.ds(base, CHROWS)], sem_w)
            nxt = j + NBUF
            if nxt < CH:
                wr[j].wait()
                rd[nxt] = pltpu.async_copy(y_hbm.at[idx_v.at[nxt]],
                                           bufs[nxt % NBUF], sem_r)
        for j in range(CH - NBUF, CH):
            if wr[j] is not None:
                wr[j].wait()

    return dispatch, combine


def _ffn_body(be_ref, nused_ref, xg_ref, w1_ref, b1_ref, w2_ref, b2_ref,
              y_ref):
    i = pl.program_id(0)

    @pl.when(i < nused_ref[0])
    def _():
        h = jnp.dot(xg_ref[...], w1_ref[0],
                    preferred_element_type=jnp.float32) + b1_ref[0]
        h = jnp.maximum(h, 0.0)
        y_ref[...] = jnp.dot(h, w2_ref[0],
                             preferred_element_type=jnp.float32) + b2_ref[0]


def _clamp(i, nu):
    return jnp.minimum(i, nu[0] - 1)


_ffn = pl.pallas_call(
    _ffn_body,
    grid_spec=pltpu.PrefetchScalarGridSpec(
        num_scalar_prefetch=2,
        grid=(G,),
        in_specs=[
            pl.BlockSpec((B, D_MODEL), lambda i, be, nu: (_clamp(i, nu), 0)),
            pl.BlockSpec((1, D_MODEL, D_FF),
                         lambda i, be, nu: (be[_clamp(i, nu)], 0, 0)),
            pl.BlockSpec((1, 1, D_FF),
                         lambda i, be, nu: (be[_clamp(i, nu)], 0, 0)),
            pl.BlockSpec((1, D_FF, D_MODEL),
                         lambda i, be, nu: (be[_clamp(i, nu)], 0, 0)),
            pl.BlockSpec((1, 1, D_MODEL),
                         lambda i, be, nu: (be[_clamp(i, nu)], 0, 0)),
        ],
        out_specs=pl.BlockSpec((B, D_MODEL),
                               lambda i, be, nu: (_clamp(i, nu), 0)),
    ),
    out_shape=jax.ShapeDtypeStruct((G * B, D_MODEL), jnp.float32),
)


def kernel(x, Wr, br, W1, b1, W2, b2):
    orig_shape = x.shape
    x_flat = x.reshape(-1, D_MODEL)
    p, be, nused = _route(x_flat, Wr, br.reshape(1, N_EXP))
    p3 = p.reshape(NW, CH, CHROWS)
    dispatch, combine = _sc_kernels()
    xg = dispatch(x_flat, p3)
    y = _ffn(be.reshape(G), nused.reshape(1), xg, W1,
             b1.reshape(N_EXP, 1, D_FF), W2, b2.reshape(N_EXP, 1, D_MODEL))
    out = combine(y, p3)
    return out.reshape(orig_shape)


# speedup vs baseline: 1.4367x; 1.4367x over previous
import functools

import jax
import jax.numpy as jnp
from jax import lax
from jax.experimental import pallas as pl
from jax.experimental.pallas import tpu as pltpu
from jax.experimental.pallas import tpu_sc as plsc

D_MODEL = 768
D_FF = 3072
N_EXP = 64
N_TOK = 8192
TB = 512
N_TB = N_TOK // TB
B = 256
G = N_TOK // B + N_EXP
NW = 32
ROWS_W = N_TOK // NW
CH = 8
CHROWS = ROWS_W // CH
NBUF = 4


def _route_body(x_ref, wr_ref, br_ref, p_ref, be_ref, nu_ref,
                cnt_s, eid_s, rank_s):
    ph = pl.program_id(0)
    j = pl.program_id(1)

    @pl.when(jnp.logical_and(ph == 0, j == 0))
    def _():
        cnt_s[...] = jnp.zeros_like(cnt_s)

    @pl.when(ph == 0)
    def _():
        logits = jnp.dot(x_ref[...], wr_ref[...],
                         preferred_element_type=jnp.float32) + br_ref[...]
        m = jnp.max(logits, axis=1, keepdims=True)
        lane = lax.broadcasted_iota(jnp.int32, (TB, N_EXP), 1)
        eid = jnp.min(jnp.where(logits == m, lane, N_EXP), axis=1,
                      keepdims=True)
        onehot = (eid == lane).astype(jnp.float32)
        row = lax.broadcasted_iota(jnp.int32, (TB, TB), 0)
        col = lax.broadcasted_iota(jnp.int32, (TB, TB), 1)
        ltri = (col < row).astype(jnp.float32)
        prev = cnt_s[...]
        cum = jnp.dot(ltri, onehot, preferred_element_type=jnp.float32) + prev
        rank = jnp.sum(cum * onehot, axis=1, keepdims=True)
        cnt_s[...] = prev + jnp.sum(onehot, axis=0, keepdims=True)
        eid_s[pl.ds(j * TB, TB), :] = eid
        rank_s[pl.ds(j * TB, TB), :] = rank.astype(jnp.int32)

    @pl.when(ph == 1)
    def _():
        cnt = cnt_s[...].astype(jnp.int32)
        nblk = lax.shift_right_logical(cnt + (B - 1),
                                       B.bit_length() - 1)
        ur = lax.broadcasted_iota(jnp.int32, (N_EXP, N_EXP), 0)
        uc = lax.broadcasted_iota(jnp.int32, (N_EXP, N_EXP), 1)
        utri = (ur < uc).astype(jnp.float32)
        bstart = jnp.dot(nblk.astype(jnp.float32), utri,
                         preferred_element_type=jnp.float32)
        eid = eid_s[pl.ds(j * TB, TB), :]
        lane = lax.broadcasted_iota(jnp.int32, (TB, N_EXP), 1)
        onehot = (eid == lane).astype(jnp.float32)
        slot = jnp.sum(onehot * (bstart * B), axis=1, keepdims=True)
        p_ref[...] = slot.astype(jnp.int32) + rank_s[pl.ds(j * TB, TB), :]

        @pl.when(j == 0)
        def _():
            gcol = lax.broadcasted_iota(jnp.int32, (G, N_EXP), 0)
            nge = jnp.sum((gcol >= bstart.astype(jnp.int32)).astype(jnp.int32),
                          axis=1, keepdims=True)
            be_ref[...] = nge - 1
            nu_ref[...] = jnp.sum(nblk, axis=1, keepdims=True)


_route = pl.pallas_call(
    _route_body,
    grid=(2, N_TB),
    in_specs=[
        pl.BlockSpec((TB, D_MODEL),
                     lambda ph, j: (jnp.where(ph == 0, j, N_TB - 1), 0)),
        pl.BlockSpec((D_MODEL, N_EXP), lambda ph, j: (0, 0)),
        pl.BlockSpec((1, N_EXP), lambda ph, j: (0, 0)),
    ],
    out_specs=[
        pl.BlockSpec((TB, 1), lambda ph, j: (j, 0)),
        pl.BlockSpec((G, 1), lambda ph, j: (0, 0)),
        pl.BlockSpec((1, 1), lambda ph, j: (0, 0)),
    ],
    out_shape=[
        jax.ShapeDtypeStruct((N_TOK, 1), jnp.int32),
        jax.ShapeDtypeStruct((G, 1), jnp.int32),
        jax.ShapeDtypeStruct((1, 1), jnp.int32),
    ],
    scratch_shapes=[
        pltpu.VMEM((1, N_EXP), jnp.float32),
        pltpu.VMEM((N_TOK, 1), jnp.int32),
        pltpu.VMEM((N_TOK, 1), jnp.int32),
    ],
)


@functools.cache
def _sc_kernels():
    mesh = plsc.VectorSubcoreMesh(core_axis_name="c", subcore_axis_name="s")
    scratch = (
        [pltpu.VMEM((CH, CHROWS), jnp.int32)]
        + [pltpu.VMEM((CHROWS, D_MODEL), jnp.float32) for _ in range(NBUF)]
        + [pltpu.SemaphoreType.DMA, pltpu.SemaphoreType.DMA]
    )


    @functools.partial(
        pl.kernel,
        mesh=mesh,
        out_type=jax.ShapeDtypeStruct((G * B, D_MODEL), jnp.float32),
        scratch_types=scratch,
    )
    def dispatch(x_hbm, p_hbm, xg_hbm, idx_v, *rest):
        bufs, (sem_r, sem_w) = rest[:NBUF], rest[NBUF:]
        wid = lax.axis_index("s") * 2 + lax.axis_index("c")
        pltpu.sync_copy(p_hbm.at[wid], idx_v)
        rd = [None] * CH
        wr = [None] * CH
        for j in range(min(NBUF, CH)):
            base = wid * ROWS_W + j * CHROWS
            rd[j] = pltpu.async_copy(x_hbm.at[pl.ds(base, CHROWS)],
                                     bufs[j % NBUF], sem_r)
        for j in range(CH):
            rd[j].wait()
            wr[j] = pltpu.async_copy(bufs[j % NBUF], xg_hbm.at[idx_v.at[j]],
                                     sem_w)
            nxt = j + NBUF
            if nxt < CH:
                wr[j].wait()
                base = wid * ROWS_W + nxt * CHROWS
                rd[nxt] = pltpu.async_copy(x_hbm.at[pl.ds(base, CHROWS)],
                                           bufs[nxt % NBUF], sem_r)
        for j in range(CH - NBUF, CH):
            if wr[j] is not None:
                wr[j].wait()

    @functools.partial(
        pl.kernel,
        mesh=mesh,
        out_type=jax.ShapeDtypeStruct((N_TOK, D_MODEL), jnp.float32),
        scratch_types=scratch,
    )
    def combine(y_hbm, p_hbm, out_hbm, idx_v, *rest):
        bufs, (sem_r, sem_w) = rest[:NBUF], rest[NBUF:]
        wid = lax.axis_index("s") * 2 + lax.axis_index("c")
        pltpu.sync_copy(p_hbm.at[wid], idx_v)
        rd = [None] * CH
        wr = [None] * CH
        for j in range(min(NBUF, CH)):
            rd[j] = pltpu.async_copy(y_hbm.at[idx_v.at[j]], bufs[j % NBUF],
                                     sem_r)
        for j in range(CH):
            rd[j].wait()
            base = wid * ROWS_W + j * CHROWS
            wr[j] = pltpu.async_copy(bufs[j % NBUF],
                                     out_hbm.at[pl.ds(base, CHROWS)], sem_w)
            nxt = j + NBUF
            if nxt < CH:
                wr[j].wait()
                rd[nxt] = pltpu.async_copy(y_hbm.at[idx_v.at[nxt]],
                                           bufs[nxt % NBUF], sem_r)
        for j in range(CH - NBUF, CH):
            if wr[j] is not None:
                wr[j].wait()

    return dispatch, combine


def _ffn_body(be_ref, nused_ref, xg_ref, w1_ref, b1_ref, w2_ref, b2_ref,
              y_ref):
    i = pl.program_id(0)

    @pl.when(i < nused_ref[0])
    def _():
        h = jnp.dot(xg_ref[...], w1_ref[0],
                    preferred_element_type=jnp.float32) + b1_ref[0]
        h = jnp.maximum(h, 0.0)
        y_ref[...] = jnp.dot(h, w2_ref[0],
                             preferred_element_type=jnp.float32) + b2_ref[0]


def _clamp(i, nu):
    return jnp.minimum(i, nu[0] - 1)


_ffn = pl.pallas_call(
    _ffn_body,
    grid_spec=pltpu.PrefetchScalarGridSpec(
        num_scalar_prefetch=2,
        grid=(G,),
        in_specs=[
            pl.BlockSpec((B, D_MODEL), lambda i, be, nu: (_clamp(i, nu), 0)),
            pl.BlockSpec((1, D_MODEL, D_FF),
                         lambda i, be, nu: (be[_clamp(i, nu)], 0, 0)),
            pl.BlockSpec((1, 1, D_FF),
                         lambda i, be, nu: (be[_clamp(i, nu)], 0, 0)),
            pl.BlockSpec((1, D_FF, D_MODEL),
                         lambda i, be, nu: (be[_clamp(i, nu)], 0, 0)),
            pl.BlockSpec((1, 1, D_MODEL),
                         lambda i, be, nu: (be[_clamp(i, nu)], 0, 0)),
        ],
        out_specs=pl.BlockSpec((B, D_MODEL),
                               lambda i, be, nu: (_clamp(i, nu), 0)),
    ),
    out_shape=jax.ShapeDtypeStruct((G * B, D_MODEL), jnp.float32),
)


def kernel(x, Wr, br, W1, b1, W2, b2):
    orig_shape = x.shape
    x_flat = x.reshape(-1, D_MODEL)
    p, be, nused = _route(x_flat, Wr, br.reshape(1, N_EXP))
    p3 = p.reshape(NW, CH, CHROWS)
    dispatch, combine = _sc_kernels()
    xg = dispatch(x_flat, p3)
    y = _ffn(be.reshape(G), nused.reshape(1), xg, W1,
             b1.reshape(N_EXP, 1, D_FF), W2, b2.reshape(N_EXP, 1, D_MODEL))
    out = combine(y, p3)
    return out.reshape(orig_shape)
